# baseline (speedup 1.0000x reference)
import functools

import jax
import jax.numpy as jnp
from jax import lax
from jax.experimental import pallas as pl
from jax.experimental.pallas import tpu as pltpu
from jax.experimental.pallas import tpu_sc as plsc

_NC = 2
_NS = 16
_CHUNK = 128


def _round_up(v, m):
    return (v + m - 1) // m * m


_IB = 2
_ZB = 128


def _make_sc_segment_sum(n_pad, d_tot, chunks_per_worker):
    nw = _NC * _NS
    rows_per_tile = n_pad // _NS
    epw = chunks_per_worker * _CHUNK
    n_grp = chunks_per_worker // _IB
    mesh = plsc.VectorSubcoreMesh(core_axis_name="c", subcore_axis_name="s")

    @functools.partial(
        pl.kernel,
        out_type=jax.ShapeDtypeStruct((_NC * n_pad, d_tot), jnp.float32),
        mesh=mesh,
        compiler_params=pltpu.CompilerParams(use_tc_tiling_on_sc=False),
        scratch_types=[
            pltpu.VMEM((_CHUNK,), jnp.int32),
            pltpu.VMEM((_CHUNK,), jnp.int32),
            pltpu.VMEM((_CHUNK, d_tot), jnp.float32),
            pltpu.VMEM_SHARED((n_pad, d_tot), jnp.float32),
            pltpu.SemaphoreType.DMA,
        ],
    )
    def sc_kernel(rows_hbm, src_hbm, dst_hbm, zeros_hbm, out_hbm,
                  src_v, dst_v, gat_v, acc_sh, sem):
        cid = lax.axis_index("c")
        sid = lax.axis_index("s")
        wid = sid * _NC + cid
        r0 = sid * rows_per_tile
        n_sub = rows_per_tile // _ZB
        pltpu.sync_copy(zeros_hbm, gat_v.at[pl.ds(0, _ZB)])
        for j in range(n_sub):
            pltpu.sync_copy(
                gat_v.at[pl.ds(0, _ZB)], acc_sh.at[pl.ds(r0 + j * _ZB, _ZB)])
        plsc.subcore_barrier()

        def body(i, carry):
            off = wid * epw + i * _CHUNK
            pltpu.sync_copy(src_hbm.at[pl.ds(off, _CHUNK)], src_v)
            pltpu.sync_copy(dst_hbm.at[pl.ds(off, _CHUNK)], dst_v)
            pltpu.async_copy(rows_hbm.at[src_v], gat_v, sem).wait()
            pltpu.sync_copy(gat_v, acc_sh.at[dst_v], add=True)
            return carry

        lax.fori_loop(0, chunks_per_worker, body, 0)
        plsc.subcore_barrier()
        for j in range(n_sub):
            pltpu.sync_copy(
                acc_sh.at[pl.ds(r0 + j * _ZB, _ZB)], gat_v.at[pl.ds(0, _ZB)])
            pltpu.sync_copy(
                gat_v.at[pl.ds(0, _ZB)],
                out_hbm.at[pl.ds(cid * n_pad + r0 + j * _ZB, _ZB)])

    return sc_kernel


def _root_mm_body(x_ref, w_ref, b_ref, o_ref, *, d):
    o_ref[...] = (
        jnp.dot(x_ref[:, :d], w_ref[...], preferred_element_type=jnp.float32)
        + b_ref[...])


def _tca_body(res1_ref, r_ref, s_ref, w1n_ref, w2n_ref, h_ref, p_ref,
              *, blk, d):
    i = pl.program_id(0)
    s = s_ref[0] + s_ref[1]
    deg = jnp.maximum(s[:, d:d + 1], 1.0)
    agg = s[:, :d] / deg
    z = r_ref[...] + jnp.dot(agg, w1n_ref[...],
                             preferred_element_type=jnp.float32)
    rows = i * blk + lax.broadcasted_iota(jnp.int32, (blk, 1), 0)
    h = jnp.where(rows < res1_ref[0], jnp.maximum(z, 0.0), 0.0)
    h_ref[...] = h
    p = jnp.dot(h, w2n_ref[...], preferred_element_type=jnp.float32)
    p_ref[...] = jnp.concatenate(
        [p, jnp.ones((blk, 16), jnp.float32)], axis=1)


def _tcb_body(res2_ref, q_ref, s_ref, o_ref, *, blk, c):
    i = pl.program_id(0)
    s = s_ref[0] + s_ref[1]
    agg = s[:, :c] / jnp.maximum(s[:, c:c + 1], 1.0)
    z = q_ref[...] + agg
    rows = i * blk + lax.broadcasted_iota(jnp.int32, (blk, 1), 0)
    z = jnp.where(rows < res2_ref[0], z, 0.0)
    m = jnp.max(z, axis=1, keepdims=True)
    e = jnp.exp(z - m)
    o_ref[...] = z - m - jnp.log(jnp.sum(e, axis=1, keepdims=True))


def kernel(x, edge_index1, res_size1, edge_index2, res_size2,
           W1_root, W1_neigh, b1, W2_root, W2_neigh, b2):
    n, d = x.shape
    hdim = W1_root.shape[1]
    cdim = W2_root.shape[1]
    e = edge_index1.shape[1]
    blk = 512
    n_pad = _round_up(n + 1, blk)
    nw = _NC * _NS
    cpw = _round_up(-(-e // (nw * _CHUNK)), _IB)
    e_pad = cpw * nw * _CHUNK
    d1 = d + 16
    d2 = cdim + 16

    xpad = jnp.pad(
        jnp.concatenate([x, jnp.ones((n, 16), jnp.float32)], axis=1),
        ((0, n_pad - n), (0, 0)))
    ei1 = edge_index1.astype(jnp.int32)
    ei2 = edge_index2.astype(jnp.int32)
    src1 = jnp.pad(ei1[0], (0, e_pad - e))
    dst1 = jnp.pad(ei1[1], (0, e_pad - e), constant_values=n)
    src2 = jnp.pad(ei2[0], (0, e_pad - e))
    dst2 = jnp.pad(ei2[1], (0, e_pad - e), constant_values=n)
    z1 = jnp.zeros((_ZB, d1), jnp.float32)
    z2 = jnp.zeros((_ZB, d2), jnp.float32)
    res1 = jnp.asarray(res_size1, jnp.int32).reshape(1)
    res2 = jnp.asarray(res_size2, jnp.int32).reshape(1)

    grid = (n_pad // blk,)

    s1 = _make_sc_segment_sum(n_pad, d1, cpw)(xpad, src1, dst1, z1)
    s1 = s1.reshape(_NC, n_pad, d1)

    r1 = pl.pallas_call(
        functools.partial(_root_mm_body, d=d),
        grid=grid,
        in_specs=[
            pl.BlockSpec((blk, d1), lambda i: (i, 0)),
            pl.BlockSpec((d, hdim), lambda i: (0, 0)),
            pl.BlockSpec((1, hdim), lambda i: (0, 0)),
        ],
        out_specs=pl.BlockSpec((blk, hdim), lambda i: (i, 0)),
        out_shape=jax.ShapeDtypeStruct((n_pad, hdim), jnp.float32),
    )(xpad, W1_root, b1.reshape(1, hdim))

    h, p = pl.pallas_call(
        functools.partial(_tca_body, blk=blk, d=d),
        grid=grid,
        in_specs=[
            pl.BlockSpec(memory_space=pltpu.SMEM),
            pl.BlockSpec((blk, hdim), lambda i: (i, 0)),
            pl.BlockSpec((_NC, blk, d1), lambda i: (0, i, 0)),
            pl.BlockSpec((d, hdim), lambda i: (0, 0)),
            pl.BlockSpec((hdim, cdim), lambda i: (0, 0)),
        ],
        out_specs=[
            pl.BlockSpec((blk, hdim), lambda i: (i, 0)),
            pl.BlockSpec((blk, d2), lambda i: (i, 0)),
        ],
        out_shape=[
            jax.ShapeDtypeStruct((n_pad, hdim), jnp.float32),
            jax.ShapeDtypeStruct((n_pad, d2), jnp.float32),
        ],
    )(res1, r1, s1, W1_neigh, W2_neigh)

    s2 = _make_sc_segment_sum(n_pad, d2, cpw)(p, src2, dst2, z2)
    s2 = s2.reshape(_NC, n_pad, d2)

    q = pl.pallas_call(
        functools.partial(_root_mm_body, d=hdim),
        grid=grid,
        in_specs=[
            pl.BlockSpec((blk, hdim), lambda i: (i, 0)),
            pl.BlockSpec((hdim, cdim), lambda i: (0, 0)),
            pl.BlockSpec((1, cdim), lambda i: (0, 0)),
        ],
        out_specs=pl.BlockSpec((blk, cdim), lambda i: (i, 0)),
        out_shape=jax.ShapeDtypeStruct((n_pad, cdim), jnp.float32),
    )(h, W2_root, b2.reshape(1, cdim))

    out = pl.pallas_call(
        functools.partial(_tcb_body, blk=blk, c=cdim),
        grid=grid,
        in_specs=[
            pl.BlockSpec(memory_space=pltpu.SMEM),
            pl.BlockSpec((blk, cdim), lambda i: (i, 0)),
            pl.BlockSpec((_NC, blk, d2), lambda i: (0, i, 0)),
        ],
        out_specs=pl.BlockSpec((blk, cdim), lambda i: (i, 0)),
        out_shape=jax.ShapeDtypeStruct((n_pad, cdim), jnp.float32),
    )(res2, q, s2)

    return out[:n]


# speedup vs baseline: 1.5510x; 1.5510x over previous
import functools

import jax
import jax.numpy as jnp
from jax import lax
from jax.experimental import pallas as pl
from jax.experimental.pallas import tpu as pltpu
from jax.experimental.pallas import tpu_sc as plsc

_NC = 2
_NS = 16
_CHUNK = 128


def _round_up(v, m):
    return (v + m - 1) // m * m


def _make_sc_segment_sum(n_pad, d_tot, chunks_per_worker):
    nw = _NC * _NS
    rows_per_tile = n_pad // _NS
    epw = chunks_per_worker * _CHUNK
    mesh = plsc.VectorSubcoreMesh(core_axis_name="c", subcore_axis_name="s")

    @functools.partial(
        pl.kernel,
        out_type=jax.ShapeDtypeStruct((_NC * n_pad, d_tot), jnp.float32),
        mesh=mesh,
        compiler_params=pltpu.CompilerParams(use_tc_tiling_on_sc=False),
        scratch_types=[
            pltpu.VMEM((2, _CHUNK), jnp.int32),
            pltpu.VMEM((_CHUNK, d_tot), jnp.float32),
            pltpu.VMEM_SHARED((n_pad, d_tot), jnp.float32),
            pltpu.SemaphoreType.DMA,
        ],
    )
    def sc_kernel(rows_hbm, edges_hbm, zeros_hbm, out_hbm,
                  idx_v, gat_v, acc_sh, sem):
        cid = lax.axis_index("c")
        sid = lax.axis_index("s")
        wid = sid * _NC + cid
        r0 = sid * rows_per_tile
        n_sub = rows_per_tile // _CHUNK
        pltpu.sync_copy(zeros_hbm, gat_v)
        for j in range(n_sub):
            pltpu.sync_copy(
                gat_v, acc_sh.at[pl.ds(r0 + j * _CHUNK, _CHUNK)])
        plsc.subcore_barrier()

        def body(i, carry):
            off = wid * epw + i * _CHUNK
            pltpu.sync_copy(edges_hbm.at[:, pl.ds(off, _CHUNK)], idx_v)
            pltpu.async_copy(rows_hbm.at[idx_v.at[0]], gat_v, sem).wait()
            pltpu.sync_copy(gat_v, acc_sh.at[idx_v.at[1]], add=True)
            return carry

        lax.fori_loop(0, chunks_per_worker, body, 0)
        plsc.subcore_barrier()
        for j in range(n_sub):
            pltpu.sync_copy(
                acc_sh.at[pl.ds(r0 + j * _CHUNK, _CHUNK)], gat_v)
            pltpu.sync_copy(
                gat_v,
                out_hbm.at[pl.ds(cid * n_pad + r0 + j * _CHUNK, _CHUNK)])

    return sc_kernel


def _tca_body(res1_ref, x_ref, s_ref, w1r_ref, w1n_ref, b1_ref, w2n_ref,
              h_ref, p_ref, *, blk, d):
    i = pl.program_id(0)
    s = s_ref[0] + s_ref[1]
    deg = jnp.maximum(s[:, d:d + 1], 1.0)
    agg = s[:, :d] / deg
    z = (jnp.dot(x_ref[:, :d], w1r_ref[...], preferred_element_type=jnp.float32)
         + jnp.dot(agg, w1n_ref[...], preferred_element_type=jnp.float32)
         + b1_ref[...])
    rows = i * blk + lax.broadcasted_iota(jnp.int32, (blk, 1), 0)
    h = jnp.where(rows < res1_ref[0], jnp.maximum(z, 0.0), 0.0)
    h_ref[...] = h
    p = jnp.dot(h, w2n_ref[...], preferred_element_type=jnp.float32)
    p_ref[...] = jnp.concatenate(
        [p, jnp.ones((blk, 16), jnp.float32)], axis=1)


def _tcb_body(res2_ref, h_ref, s_ref, w2r_ref, b2_ref, o_ref, *, blk, c):
    i = pl.program_id(0)
    s = s_ref[0] + s_ref[1]
    agg = s[:, :c] / jnp.maximum(s[:, c:c + 1], 1.0)
    z = (jnp.dot(h_ref[...], w2r_ref[...], preferred_element_type=jnp.float32)
         + agg + b2_ref[...])
    rows = i * blk + lax.broadcasted_iota(jnp.int32, (blk, 1), 0)
    z = jnp.where(rows < res2_ref[0], z, 0.0)
    m = jnp.max(z, axis=1, keepdims=True)
    e = jnp.exp(z - m)
    o_ref[...] = z - m - jnp.log(jnp.sum(e, axis=1, keepdims=True))


def kernel(x, edge_index1, res_size1, edge_index2, res_size2,
           W1_root, W1_neigh, b1, W2_root, W2_neigh, b2):
    n, d = x.shape
    hdim = W1_root.shape[1]
    cdim = W2_root.shape[1]
    e = edge_index1.shape[1]
    blk = 512
    n_pad = _round_up(n + 1, blk)
    nw = _NC * _NS
    cpw = -(-e // (nw * _CHUNK))
    e_pad = cpw * nw * _CHUNK
    d1 = d + 16
    d2 = cdim + 16

    xpad = jnp.pad(
        jnp.concatenate([x, jnp.ones((n, 16), jnp.float32)], axis=1),
        ((0, n_pad - n), (0, 0)))
    ei1 = edge_index1.astype(jnp.int32)
    ei2 = edge_index2.astype(jnp.int32)
    edges1 = jnp.stack([jnp.pad(ei1[0], (0, e_pad - e)),
                        jnp.pad(ei1[1], (0, e_pad - e), constant_values=n)])
    edges2 = jnp.stack([jnp.pad(ei2[0], (0, e_pad - e)),
                        jnp.pad(ei2[1], (0, e_pad - e), constant_values=n)])
    z1 = jnp.zeros((_CHUNK, d1), jnp.float32)
    z2 = jnp.zeros((_CHUNK, d2), jnp.float32)
    res1 = jnp.asarray(res_size1, jnp.int32).reshape(1)
    res2 = jnp.asarray(res_size2, jnp.int32).reshape(1)

    s1 = _make_sc_segment_sum(n_pad, d1, cpw)(xpad, edges1, z1)
    s1 = s1.reshape(_NC, n_pad, d1)

    grid = (n_pad // blk,)
    h, p = pl.pallas_call(
        functools.partial(_tca_body, blk=blk, d=d),
        grid=grid,
        in_specs=[
            pl.BlockSpec(memory_space=pltpu.SMEM),
            pl.BlockSpec((blk, d1), lambda i: (i, 0)),
            pl.BlockSpec((_NC, blk, d1), lambda i: (0, i, 0)),
            pl.BlockSpec((d, hdim), lambda i: (0, 0)),
            pl.BlockSpec((d, hdim), lambda i: (0, 0)),
            pl.BlockSpec((1, hdim), lambda i: (0, 0)),
            pl.BlockSpec((hdim, cdim), lambda i: (0, 0)),
        ],
        out_specs=[
            pl.BlockSpec((blk, hdim), lambda i: (i, 0)),
            pl.BlockSpec((blk, d2), lambda i: (i, 0)),
        ],
        out_shape=[
            jax.ShapeDtypeStruct((n_pad, hdim), jnp.float32),
            jax.ShapeDtypeStruct((n_pad, d2), jnp.float32),
        ],
    )(res1, xpad, s1, W1_root, W1_neigh, b1.reshape(1, hdim), W2_neigh)

    s2 = _make_sc_segment_sum(n_pad, d2, cpw)(p, edges2, z2)
    s2 = s2.reshape(_NC, n_pad, d2)

    out = pl.pallas_call(
        functools.partial(_tcb_body, blk=blk, c=cdim),
        grid=grid,
        in_specs=[
            pl.BlockSpec(memory_space=pltpu.SMEM),
            pl.BlockSpec((blk, hdim), lambda i: (i, 0)),
            pl.BlockSpec((_NC, blk, d2), lambda i: (0, i, 0)),
            pl.BlockSpec((hdim, cdim), lambda i: (0, 0)),
            pl.BlockSpec((1, cdim), lambda i: (0, 0)),
        ],
        out_specs=pl.BlockSpec((blk, cdim), lambda i: (i, 0)),
        out_shape=jax.ShapeDtypeStruct((n_pad, cdim), jnp.float32),
    )(res2, h, s2, W2_root, b2.reshape(1, cdim))

    return out[:n]
